# baseline (speedup 1.0000x reference)
import jax
import jax.numpy as jnp
from jax.experimental import pallas as pl
from jax.experimental.pallas import tpu as pltpu

_BM = 200


def _body(x1_ref, x2_ref, wg_ref, adj_ref, bias_ref, a_ref, wlin_ref,
          blin_ref, out_ref, h_ref):
    d_h = wlin_ref.shape[0]
    i = pl.program_id(0)

    @pl.when(i == 0)
    def _init():
        dn = (((1,), (1,)), ((), ()))
        h_ref[:, :d_h] = jax.lax.dot_general(
            x1_ref[...], wg_ref[...], dn,
            preferred_element_type=jnp.float32).astype(jnp.bfloat16)
        h_ref[:, d_h:] = jax.lax.dot_general(
            x2_ref[...], wg_ref[...], dn,
            preferred_element_type=jnp.float32).astype(jnp.bfloat16)

    acc = jnp.dot(adj_ref[...].astype(jnp.bfloat16), h_ref[...],
                  preferred_element_type=jnp.float32)
    bias2 = jnp.concatenate([bias_ref[...], bias_ref[...]], axis=1)
    o = acc + bias2
    a = a_ref[0, 0]
    o = jnp.where(o >= 0.0, o, a * o)
    s = jnp.sum(wlin_ref[...], axis=0, keepdims=True)
    s2 = jnp.concatenate([s, s], axis=1)
    w = o * s2
    bsum = jnp.sum(blin_ref[...])
    z1 = jnp.sum(w[:, :d_h], axis=1, keepdims=True) + bsum
    z2 = jnp.sum(w[:, d_h:], axis=1, keepdims=True) + bsum
    out_ref[...] = jnp.concatenate([z1, z2], axis=1)


def kernel(x_1, x_2, adj, sparse, W_gcn, bias_gcn, prelu_a, W_lin, b_lin):
    n, d_in = x_1.shape
    d_h = W_gcn.shape[0]

    bias2d = bias_gcn.reshape(1, d_h)
    a2d = jnp.asarray(prelu_a, jnp.float32).reshape(1, 1)
    blin2d = b_lin.reshape(1, d_h)

    z = pl.pallas_call(
        _body,
        grid=(n // _BM,),
        in_specs=[
            pl.BlockSpec((n, d_in), lambda i: (0, 0)),
            pl.BlockSpec((n, d_in), lambda i: (0, 0)),
            pl.BlockSpec((d_h, d_in), lambda i: (0, 0)),
            pl.BlockSpec((_BM, n), lambda i: (i, 0)),
            pl.BlockSpec((1, d_h), lambda i: (0, 0)),
            pl.BlockSpec((1, 1), lambda i: (0, 0)),
            pl.BlockSpec((d_h, d_h), lambda i: (0, 0)),
            pl.BlockSpec((1, d_h), lambda i: (0, 0)),
        ],
        out_specs=pl.BlockSpec((_BM, 2), lambda i: (i, 0)),
        out_shape=jax.ShapeDtypeStruct((n, 2), jnp.float32),
        scratch_shapes=[pltpu.VMEM((n, 2 * d_h), jnp.bfloat16)],
        compiler_params=pltpu.CompilerParams(
            dimension_semantics=("arbitrary",)),
    )(x_1, x_2, W_gcn, adj, bias2d, a2d, W_lin, blin2d)

    return z.T.reshape(2 * n)


# speedup vs baseline: 1.0258x; 1.0258x over previous
import jax
import jax.numpy as jnp
from jax.experimental import pallas as pl
from jax.experimental.pallas import tpu as pltpu

_BM = 400


def _body(x1_ref, x2_ref, wg_ref, adj_ref, bias_ref, a_ref, wlin_ref,
          blin_ref, out_ref, h_ref):
    d_h = wlin_ref.shape[0]
    i = pl.program_id(0)

    @pl.when(i == 0)
    def _init():
        dn = (((1,), (1,)), ((), ()))
        h_ref[:, :d_h] = jax.lax.dot_general(
            x1_ref[...], wg_ref[...], dn,
            preferred_element_type=jnp.float32).astype(jnp.bfloat16)
        h_ref[:, d_h:] = jax.lax.dot_general(
            x2_ref[...], wg_ref[...], dn,
            preferred_element_type=jnp.float32).astype(jnp.bfloat16)

    acc = jnp.dot(adj_ref[...].astype(jnp.bfloat16), h_ref[...],
                  preferred_element_type=jnp.float32)
    bias2 = jnp.concatenate([bias_ref[...], bias_ref[...]], axis=1)
    o = acc + bias2
    a = a_ref[0, 0]
    o = jnp.where(o >= 0.0, o, a * o)
    s = jnp.sum(wlin_ref[...], axis=0, keepdims=True)
    s2 = jnp.concatenate([s, s], axis=1)
    w = o * s2
    bsum = jnp.sum(blin_ref[...])
    z1 = jnp.sum(w[:, :d_h], axis=1, keepdims=True) + bsum
    z2 = jnp.sum(w[:, d_h:], axis=1, keepdims=True) + bsum
    out_ref[...] = jnp.concatenate([z1, z2], axis=1)


def kernel(x_1, x_2, adj, sparse, W_gcn, bias_gcn, prelu_a, W_lin, b_lin):
    n, d_in = x_1.shape
    d_h = W_gcn.shape[0]

    bias2d = bias_gcn.reshape(1, d_h)
    a2d = jnp.asarray(prelu_a, jnp.float32).reshape(1, 1)
    blin2d = b_lin.reshape(1, d_h)

    z = pl.pallas_call(
        _body,
        grid=(n // _BM,),
        in_specs=[
            pl.BlockSpec((n, d_in), lambda i: (0, 0)),
            pl.BlockSpec((n, d_in), lambda i: (0, 0)),
            pl.BlockSpec((d_h, d_in), lambda i: (0, 0)),
            pl.BlockSpec((_BM, n), lambda i: (i, 0)),
            pl.BlockSpec((1, d_h), lambda i: (0, 0)),
            pl.BlockSpec((1, 1), lambda i: (0, 0)),
            pl.BlockSpec((d_h, d_h), lambda i: (0, 0)),
            pl.BlockSpec((1, d_h), lambda i: (0, 0)),
        ],
        out_specs=pl.BlockSpec((_BM, 2), lambda i: (i, 0)),
        out_shape=jax.ShapeDtypeStruct((n, 2), jnp.float32),
        scratch_shapes=[pltpu.VMEM((n, 2 * d_h), jnp.bfloat16)],
        compiler_params=pltpu.CompilerParams(
            dimension_semantics=("arbitrary",)),
    )(x_1, x_2, W_gcn, adj, bias2d, a2d, W_lin, blin2d)

    return z.T.reshape(2 * n)
